# baseline (speedup 1.0000x reference)
import functools

import jax
import jax.numpy as jnp
from jax import lax
from jax.experimental import pallas as pl
from jax.experimental.pallas import tpu as pltpu
from jax.experimental.pallas import tpu_sc as plsc

N_NODES = 10000
N_EDGES = 160000
FEAT = 256
HALF = 128
NSUB = 16
EDGES_PER_TILE = N_EDGES // NSUB
CHUNK = 125
NCHUNK = EDGES_PER_TILE // CHUNK
NPASS = 2
CPP = NCHUNK // NPASS
ROWS_PER_TILE = 624
TAIL_BASE = NSUB * ROWS_PER_TILE
TAIL = N_NODES - TAIL_BASE
CNT_W = 128
BLK = 1000


def _sc_aggregate(xt, dst_r, src_r, zrow, ones):
    mesh = plsc.VectorSubcoreMesh(core_axis_name="c", subcore_axis_name="s",
                                  num_cores=2, num_subcores=NSUB)

    @functools.partial(
        pl.kernel,
        out_type=(
            jax.ShapeDtypeStruct((2, N_NODES, HALF), jnp.float32),
            jax.ShapeDtypeStruct((2, N_NODES, CNT_W), jnp.float32),
        ),
        mesh=mesh,
        scratch_types=[
            pltpu.VMEM((CPP, CHUNK), jnp.int32),
            pltpu.VMEM((CPP, CHUNK), jnp.int32),
            pltpu.VMEM((CHUNK, HALF), jnp.float32),
            pltpu.VMEM((CHUNK, HALF), jnp.float32),
            pltpu.VMEM_SHARED((N_NODES, HALF), jnp.float32),
            pltpu.SemaphoreType.DMA,
            pltpu.SemaphoreType.DMA,
        ],
    )
    def agg_kernel(x_hbm, dst_hbm, src_hbm, zrow_hbm, ones_hbm,
                   sums_hbm, cnt_hbm, src_v, dst_v, rows0_v, rows1_v, acc,
                   sem0, sem1):
        c = lax.axis_index("c")
        s = lax.axis_index("s")
        base_row = s * ROWS_PER_TILE

        def zero_acc():
            pltpu.sync_copy(zrow_hbm, acc.at[pl.ds(base_row, ROWS_PER_TILE)])

            @pl.when(s == NSUB - 1)
            def _():
                pltpu.sync_copy(zrow_hbm.at[pl.ds(0, TAIL)],
                                acc.at[pl.ds(TAIL_BASE, TAIL)])

        def write_out(out_hbm):
            pltpu.sync_copy(acc.at[pl.ds(base_row, ROWS_PER_TILE)],
                            out_hbm.at[c].at[pl.ds(base_row, ROWS_PER_TILE)])

            @pl.when(s == NSUB - 1)
            def _():
                pltpu.sync_copy(acc.at[pl.ds(TAIL_BASE, TAIL)],
                                out_hbm.at[c].at[pl.ds(TAIL_BASE, TAIL)])

        zero_acc()
        plsc.subcore_barrier()

        x_view = x_hbm.at[c]

        @pl.loop(0, NPASS)
        def _(p):
            pltpu.sync_copy(src_hbm.at[s].at[pl.ds(p * CPP, CPP)], src_v)
            pltpu.sync_copy(dst_hbm.at[s].at[pl.ds(p * CPP, CPP)], dst_v)
            pltpu.async_copy(x_view.at[src_v.at[0]], rows0_v, sem0)

            @pl.loop(0, CPP // 2)
            def _(k):
                j0 = 2 * k
                g1 = pltpu.async_copy(x_view.at[src_v.at[j0 + 1]], rows1_v,
                                      sem1)
                pltpu.make_async_copy(x_view.at[src_v.at[j0]], rows0_v,
                                      sem0).wait()
                pltpu.sync_copy(rows0_v, acc.at[dst_v.at[j0]], add=True)

                @pl.when(j0 + 2 < CPP)
                def _():
                    pltpu.async_copy(x_view.at[src_v.at[j0 + 2]], rows0_v,
                                     sem0)

                g1.wait()
                pltpu.sync_copy(rows1_v, acc.at[dst_v.at[j0 + 1]], add=True)

        plsc.subcore_barrier()
        write_out(sums_hbm)
        plsc.subcore_barrier()

        zero_acc()
        pltpu.sync_copy(ones_hbm, rows0_v)
        pltpu.sync_copy(dst_hbm.at[s].at[pl.ds(c * CPP, CPP)], dst_v)
        plsc.subcore_barrier()

        @pl.loop(0, CPP)
        def _(j):
            pltpu.sync_copy(rows0_v, acc.at[dst_v.at[j]], add=True)

        plsc.subcore_barrier()
        write_out(cnt_hbm)

    return agg_kernel(xt, dst_r, src_r, zrow, ones)


def _tc_nodes(features, weight):
    def body(x_ref, w_ref, o_ref):
        o_ref[...] = jnp.dot(x_ref[...], w_ref[...],
                             preferred_element_type=jnp.float32)

    return pl.pallas_call(
        body,
        grid=(N_NODES // BLK,),
        in_specs=[
            pl.BlockSpec((BLK, FEAT), lambda i: (i, 0)),
            pl.BlockSpec((FEAT, FEAT), lambda i: (0, 0)),
        ],
        out_specs=pl.BlockSpec((BLK, FEAT), lambda i: (i, 0)),
        out_shape=jax.ShapeDtypeStruct((N_NODES, FEAT), jnp.float32),
    )(features, weight)


def _tc_combine(nodes, sums2, counts, weight):
    def body(n_ref, s_ref, c_ref, w_ref, o_ref):
        agg = jnp.concatenate([s_ref[0], s_ref[1]], axis=-1)
        cnt = (c_ref[0] + c_ref[1])[:, :1]
        agg = agg / jnp.maximum(cnt, 1.0)
        msgs = jnp.dot(agg, w_ref[...], preferred_element_type=jnp.float32)
        o_ref[...] = jnp.concatenate([n_ref[...], msgs], axis=-1)

    return pl.pallas_call(
        body,
        grid=(N_NODES // BLK,),
        in_specs=[
            pl.BlockSpec((BLK, FEAT), lambda i: (i, 0)),
            pl.BlockSpec((2, BLK, HALF), lambda i: (0, i, 0)),
            pl.BlockSpec((2, BLK, CNT_W), lambda i: (0, i, 0)),
            pl.BlockSpec((FEAT, FEAT), lambda i: (0, 0)),
        ],
        out_specs=pl.BlockSpec((BLK, 2 * FEAT), lambda i: (i, 0)),
        out_shape=jax.ShapeDtypeStruct((N_NODES, 2 * FEAT), jnp.float32),
    )(nodes, sums2, counts, weight)


def _tc_split(features):
    def body(x_ref, o_ref):
        o_ref[0] = x_ref[:, :HALF]
        o_ref[1] = x_ref[:, HALF:]

    return pl.pallas_call(
        body,
        grid=(N_NODES // BLK,),
        in_specs=[pl.BlockSpec((BLK, FEAT), lambda i: (i, 0))],
        out_specs=pl.BlockSpec((2, BLK, HALF), lambda i: (0, i, 0)),
        out_shape=jax.ShapeDtypeStruct((2, N_NODES, HALF), jnp.float32),
    )(features)


def kernel(features, edge_index, weight):
    xt = _tc_split(features)
    dst_r = edge_index[0].reshape(NSUB, NCHUNK, CHUNK)
    src_r = edge_index[1].reshape(NSUB, NCHUNK, CHUNK)
    zrow = jnp.zeros((ROWS_PER_TILE, HALF), jnp.float32)
    ones = jnp.ones((CHUNK, HALF), jnp.float32)
    sums2, counts = _sc_aggregate(xt, dst_r, src_r, zrow, ones)
    nodes = _tc_nodes(features, weight)
    return _tc_combine(nodes, sums2, counts, weight)


# speedup vs baseline: 1.0165x; 1.0165x over previous
import functools

import jax
import jax.numpy as jnp
from jax import lax
from jax.experimental import pallas as pl
from jax.experimental.pallas import tpu as pltpu
from jax.experimental.pallas import tpu_sc as plsc

N_NODES = 10000
N_EDGES = 160000
FEAT = 256
HALF = 128
NSUB = 16
EDGES_PER_TILE = N_EDGES // NSUB
CHUNK = 125
NCHUNK = EDGES_PER_TILE // CHUNK
NPASS = 2
CPP = NCHUNK // NPASS
ROWS_PER_TILE = 624
TAIL_BASE = NSUB * ROWS_PER_TILE
TAIL = N_NODES - TAIL_BASE
CNT_W = 128
BLK = 1000


def _sc_aggregate(xt, dst_r, src_r, zrow, ones):
    mesh = plsc.VectorSubcoreMesh(core_axis_name="c", subcore_axis_name="s",
                                  num_cores=2, num_subcores=NSUB)

    @functools.partial(
        pl.kernel,
        out_type=(
            jax.ShapeDtypeStruct((2, N_NODES, HALF), jnp.float32),
            jax.ShapeDtypeStruct((2, N_NODES, CNT_W), jnp.float32),
        ),
        mesh=mesh,
        scratch_types=[
            pltpu.VMEM((CPP, CHUNK), jnp.int32),
            pltpu.VMEM((CPP, CHUNK), jnp.int32),
            pltpu.VMEM((CHUNK, HALF), jnp.float32),
            pltpu.VMEM((CHUNK, HALF), jnp.float32),
            pltpu.VMEM_SHARED((N_NODES, HALF), jnp.float32),
            pltpu.SemaphoreType.DMA,
            pltpu.SemaphoreType.DMA,
        ],
    )
    def agg_kernel(x_hbm, dst_hbm, src_hbm, zrow_hbm, ones_hbm,
                   sums_hbm, cnt_hbm, src_v, dst_v, rows0_v, rows1_v, acc,
                   sem0, sem1):
        c = lax.axis_index("c")
        s = lax.axis_index("s")
        base_row = s * ROWS_PER_TILE

        def zero_acc():
            pltpu.sync_copy(zrow_hbm, acc.at[pl.ds(base_row, ROWS_PER_TILE)])

            @pl.when(s == NSUB - 1)
            def _():
                pltpu.sync_copy(zrow_hbm.at[pl.ds(0, TAIL)],
                                acc.at[pl.ds(TAIL_BASE, TAIL)])

        def write_out(out_hbm):
            pltpu.sync_copy(acc.at[pl.ds(base_row, ROWS_PER_TILE)],
                            out_hbm.at[c].at[pl.ds(base_row, ROWS_PER_TILE)])

            @pl.when(s == NSUB - 1)
            def _():
                pltpu.sync_copy(acc.at[pl.ds(TAIL_BASE, TAIL)],
                                out_hbm.at[c].at[pl.ds(TAIL_BASE, TAIL)])

        zero_acc()
        plsc.subcore_barrier()

        x_view = x_hbm.at[c]

        @pl.loop(0, NPASS)
        def _(p):
            pltpu.sync_copy(src_hbm.at[s].at[pl.ds(p * CPP, CPP)], src_v)
            pltpu.sync_copy(dst_hbm.at[s].at[pl.ds(p * CPP, CPP)], dst_v)
            pltpu.async_copy(x_view.at[src_v.at[0]], rows0_v, sem0)

            @pl.loop(0, CPP // 2)
            def _(k):
                j0 = 2 * k
                g1 = pltpu.async_copy(x_view.at[src_v.at[j0 + 1]], rows1_v,
                                      sem1)
                pltpu.make_async_copy(x_view.at[src_v.at[j0]], rows0_v,
                                      sem0).wait()
                pltpu.sync_copy(rows0_v, acc.at[dst_v.at[j0]], add=True)

                @pl.when(j0 + 2 < CPP)
                def _():
                    pltpu.async_copy(x_view.at[src_v.at[j0 + 2]], rows0_v,
                                     sem0)

                g1.wait()
                pltpu.sync_copy(rows1_v, acc.at[dst_v.at[j0 + 1]], add=True)

        plsc.subcore_barrier()
        write_out(sums_hbm)
        plsc.subcore_barrier()

        zero_acc()
        pltpu.sync_copy(ones_hbm, rows0_v)
        pltpu.sync_copy(dst_hbm.at[s].at[pl.ds(c * CPP, CPP)], dst_v)
        plsc.subcore_barrier()

        @pl.loop(0, CPP)
        def _(j):
            pltpu.sync_copy(rows0_v, acc.at[dst_v.at[j]], add=True)

        plsc.subcore_barrier()
        write_out(cnt_hbm)

    return agg_kernel(xt, dst_r, src_r, zrow, ones)


def _tc_combine(features, sums2, counts, weight):
    def body(x_ref, s_ref, c_ref, w_ref, o_ref):
        w = w_ref[...]
        nodes = jnp.dot(x_ref[...], w, preferred_element_type=jnp.float32)
        agg = jnp.concatenate([s_ref[0], s_ref[1]], axis=-1)
        cnt = (c_ref[0] + c_ref[1])[:, :1]
        agg = agg / jnp.maximum(cnt, 1.0)
        msgs = jnp.dot(agg, w, preferred_element_type=jnp.float32)
        o_ref[...] = jnp.concatenate([nodes, msgs], axis=-1)

    return pl.pallas_call(
        body,
        grid=(N_NODES // BLK,),
        in_specs=[
            pl.BlockSpec((BLK, FEAT), lambda i: (i, 0)),
            pl.BlockSpec((2, BLK, HALF), lambda i: (0, i, 0)),
            pl.BlockSpec((2, BLK, CNT_W), lambda i: (0, i, 0)),
            pl.BlockSpec((FEAT, FEAT), lambda i: (0, 0)),
        ],
        out_specs=pl.BlockSpec((BLK, 2 * FEAT), lambda i: (i, 0)),
        out_shape=jax.ShapeDtypeStruct((N_NODES, 2 * FEAT), jnp.float32),
    )(features, sums2, counts, weight)


def _tc_split(features):
    def body(x_ref, o_ref):
        o_ref[0] = x_ref[:, :HALF]
        o_ref[1] = x_ref[:, HALF:]

    return pl.pallas_call(
        body,
        grid=(N_NODES // BLK,),
        in_specs=[pl.BlockSpec((BLK, FEAT), lambda i: (i, 0))],
        out_specs=pl.BlockSpec((2, BLK, HALF), lambda i: (0, i, 0)),
        out_shape=jax.ShapeDtypeStruct((2, N_NODES, HALF), jnp.float32),
    )(features)


def kernel(features, edge_index, weight):
    xt = _tc_split(features)
    dst_r = edge_index[0].reshape(NSUB, NCHUNK, CHUNK)
    src_r = edge_index[1].reshape(NSUB, NCHUNK, CHUNK)
    zrow = jnp.zeros((ROWS_PER_TILE, HALF), jnp.float32)
    ones = jnp.ones((CHUNK, HALF), jnp.float32)
    sums2, counts = _sc_aggregate(xt, dst_r, src_r, zrow, ones)
    return _tc_combine(features, sums2, counts, weight)


# speedup vs baseline: 1.0488x; 1.0318x over previous
import functools

import jax
import jax.numpy as jnp
from jax import lax
from jax.experimental import pallas as pl
from jax.experimental.pallas import tpu as pltpu
from jax.experimental.pallas import tpu_sc as plsc

N_NODES = 10000
N_EDGES = 160000
FEAT = 256
HALF = 128
NSUB = 16
EDGES_PER_TILE = N_EDGES // NSUB
CHUNK = 125
NCHUNK = EDGES_PER_TILE // CHUNK
NPASS = 2
CPP = NCHUNK // NPASS
ROWS_PER_TILE = 624
TAIL_BASE = NSUB * ROWS_PER_TILE
TAIL = N_NODES - TAIL_BASE
CNT_W = 128
CNT_OUT = 16
BLK = 1000


def _sc_aggregate(xt, dst_r, src_r, zrow, ones):
    mesh = plsc.VectorSubcoreMesh(core_axis_name="c", subcore_axis_name="s",
                                  num_cores=2, num_subcores=NSUB)

    @functools.partial(
        pl.kernel,
        out_type=(
            jax.ShapeDtypeStruct((2, N_NODES, HALF), jnp.float32),
            jax.ShapeDtypeStruct((2, N_NODES, CNT_W), jnp.float32),
        ),
        mesh=mesh,
        scratch_types=[
            pltpu.VMEM((CPP, CHUNK), jnp.int32),
            pltpu.VMEM((CPP, CHUNK), jnp.int32),
            pltpu.VMEM((CHUNK, HALF), jnp.float32),
            pltpu.VMEM((CHUNK, HALF), jnp.float32),
            pltpu.VMEM_SHARED((N_NODES, HALF), jnp.float32),
            pltpu.SemaphoreType.DMA,
            pltpu.SemaphoreType.DMA,
        ],
    )
    def agg_kernel(x_hbm, dst_hbm, src_hbm, zrow_hbm, ones_hbm,
                   sums_hbm, cnt_hbm, src_v, dst_v, rows0_v, rows1_v, acc,
                   sem0, sem1):
        c = lax.axis_index("c")
        s = lax.axis_index("s")
        base_row = s * ROWS_PER_TILE

        def zero_acc():
            pltpu.sync_copy(zrow_hbm, acc.at[pl.ds(base_row, ROWS_PER_TILE)])

            @pl.when(s == NSUB - 1)
            def _():
                pltpu.sync_copy(zrow_hbm.at[pl.ds(0, TAIL)],
                                acc.at[pl.ds(TAIL_BASE, TAIL)])

        def write_out(out_hbm):
            pltpu.sync_copy(acc.at[pl.ds(base_row, ROWS_PER_TILE)],
                            out_hbm.at[c].at[pl.ds(base_row, ROWS_PER_TILE)])

            @pl.when(s == NSUB - 1)
            def _():
                pltpu.sync_copy(acc.at[pl.ds(TAIL_BASE, TAIL)],
                                out_hbm.at[c].at[pl.ds(TAIL_BASE, TAIL)])

        zero_acc()
        plsc.subcore_barrier()

        x_view = x_hbm.at[:, pl.ds(c * HALF, HALF)]

        @pl.loop(0, NPASS)
        def _(p):
            pltpu.sync_copy(src_hbm.at[s].at[pl.ds(p * CPP, CPP)], src_v)
            pltpu.sync_copy(dst_hbm.at[s].at[pl.ds(p * CPP, CPP)], dst_v)
            pltpu.async_copy(x_view.at[src_v.at[0]], rows0_v, sem0)

            @pl.loop(0, CPP // 2)
            def _(k):
                j0 = 2 * k
                g1 = pltpu.async_copy(x_view.at[src_v.at[j0 + 1]], rows1_v,
                                      sem1)
                pltpu.make_async_copy(x_view.at[src_v.at[j0]], rows0_v,
                                      sem0).wait()
                pltpu.sync_copy(rows0_v, acc.at[dst_v.at[j0]], add=True)

                @pl.when(j0 + 2 < CPP)
                def _():
                    pltpu.async_copy(x_view.at[src_v.at[j0 + 2]], rows0_v,
                                     sem0)

                g1.wait()
                pltpu.sync_copy(rows1_v, acc.at[dst_v.at[j0 + 1]], add=True)

        plsc.subcore_barrier()
        write_out(sums_hbm)
        plsc.subcore_barrier()

        zero_acc()
        pltpu.sync_copy(ones_hbm, rows0_v)
        pltpu.sync_copy(dst_hbm.at[s].at[pl.ds(c * CPP, CPP)], dst_v)
        plsc.subcore_barrier()

        @pl.loop(0, CPP)
        def _(j):
            pltpu.sync_copy(rows0_v, acc.at[dst_v.at[j]], add=True)

        plsc.subcore_barrier()
        write_out(cnt_hbm)

    return agg_kernel(xt, dst_r, src_r, zrow, ones)


def _tc_combine(features, sums2, counts, weight):
    def body(x_ref, s_ref, c_ref, w_ref, o_ref):
        w = w_ref[...]
        nodes = jnp.dot(x_ref[...], w, preferred_element_type=jnp.float32)
        agg = jnp.concatenate([s_ref[0], s_ref[1]], axis=-1)
        cnt = (c_ref[0] + c_ref[1])[:, :1]
        agg = agg / jnp.maximum(cnt, 1.0)
        msgs = jnp.dot(agg, w, preferred_element_type=jnp.float32)
        o_ref[...] = jnp.concatenate([nodes, msgs], axis=-1)

    return pl.pallas_call(
        body,
        grid=(N_NODES // BLK,),
        in_specs=[
            pl.BlockSpec((BLK, FEAT), lambda i: (i, 0)),
            pl.BlockSpec((2, BLK, HALF), lambda i: (0, i, 0)),
            pl.BlockSpec((2, BLK, CNT_W), lambda i: (0, i, 0)),
            pl.BlockSpec((FEAT, FEAT), lambda i: (0, 0)),
        ],
        out_specs=pl.BlockSpec((BLK, 2 * FEAT), lambda i: (i, 0)),
        out_shape=jax.ShapeDtypeStruct((N_NODES, 2 * FEAT), jnp.float32),
    )(features, sums2, counts, weight)


def _tc_split(features):
    def body(x_ref, o_ref):
        o_ref[0] = x_ref[:, :HALF]
        o_ref[1] = x_ref[:, HALF:]

    return pl.pallas_call(
        body,
        grid=(N_NODES // BLK,),
        in_specs=[pl.BlockSpec((BLK, FEAT), lambda i: (i, 0))],
        out_specs=pl.BlockSpec((2, BLK, HALF), lambda i: (0, i, 0)),
        out_shape=jax.ShapeDtypeStruct((2, N_NODES, HALF), jnp.float32),
    )(features)


def kernel(features, edge_index, weight):
    xt = features
    dst_r = edge_index[0].reshape(NSUB, NCHUNK, CHUNK)
    src_r = edge_index[1].reshape(NSUB, NCHUNK, CHUNK)
    zrow = jnp.zeros((ROWS_PER_TILE, HALF), jnp.float32)
    ones = jnp.ones((CHUNK, HALF), jnp.float32)
    sums2, counts = _sc_aggregate(xt, dst_r, src_r, zrow, ones)
    return _tc_combine(features, sums2, counts, weight)


# speedup vs baseline: 1.0500x; 1.0012x over previous
import functools

import jax
import jax.numpy as jnp
from jax import lax
from jax.experimental import pallas as pl
from jax.experimental.pallas import tpu as pltpu
from jax.experimental.pallas import tpu_sc as plsc

N_NODES = 10000
N_EDGES = 160000
FEAT = 256
HALF = 128
NSUB = 16
EDGES_PER_TILE = N_EDGES // NSUB
CHUNK = 125
NCHUNK = EDGES_PER_TILE // CHUNK
NPASS = 2
CPP = NCHUNK // NPASS
ROWS_PER_TILE = 624
TAIL_BASE = NSUB * ROWS_PER_TILE
TAIL = N_NODES - TAIL_BASE
CNT_W = 128
CNT_OUT = 16
BLK = 1000


def _sc_aggregate(xt, dst_r, src_r, zrow, ones):
    mesh = plsc.VectorSubcoreMesh(core_axis_name="c", subcore_axis_name="s",
                                  num_cores=2, num_subcores=NSUB)

    @functools.partial(
        pl.kernel,
        out_type=(
            jax.ShapeDtypeStruct((2, N_NODES, HALF), jnp.float32),
            jax.ShapeDtypeStruct((2, N_NODES, CNT_W), jnp.float32),
        ),
        mesh=mesh,
        scratch_types=[
            pltpu.VMEM((CPP, CHUNK), jnp.int32),
            pltpu.VMEM((CPP, CHUNK), jnp.int32),
            pltpu.VMEM((CHUNK, HALF), jnp.float32),
            pltpu.VMEM((CHUNK, HALF), jnp.float32),
            pltpu.VMEM_SHARED((N_NODES, HALF), jnp.float32),
            pltpu.SemaphoreType.DMA,
            pltpu.SemaphoreType.DMA,
        ],
    )
    def agg_kernel(x_hbm, dst_hbm, src_hbm, zrow_hbm, ones_hbm,
                   sums_hbm, cnt_hbm, src_v, dst_v, rows0_v, rows1_v, acc,
                   sem0, sem1):
        c = lax.axis_index("c")
        s = lax.axis_index("s")
        base_row = s * ROWS_PER_TILE

        def zero_acc():
            pltpu.sync_copy(zrow_hbm, acc.at[pl.ds(base_row, ROWS_PER_TILE)])

            @pl.when(s == NSUB - 1)
            def _():
                pltpu.sync_copy(zrow_hbm.at[pl.ds(0, TAIL)],
                                acc.at[pl.ds(TAIL_BASE, TAIL)])

        def write_out(out_hbm):
            pltpu.sync_copy(acc.at[pl.ds(base_row, ROWS_PER_TILE)],
                            out_hbm.at[c].at[pl.ds(base_row, ROWS_PER_TILE)])

            @pl.when(s == NSUB - 1)
            def _():
                pltpu.sync_copy(acc.at[pl.ds(TAIL_BASE, TAIL)],
                                out_hbm.at[c].at[pl.ds(TAIL_BASE, TAIL)])

        zero_acc()
        plsc.subcore_barrier()

        x_view = x_hbm.at[:, pl.ds(c * HALF, HALF)]

        @pl.loop(0, NPASS)
        def _(p):
            pltpu.sync_copy(src_hbm.at[s].at[pl.ds(p * CPP, CPP)], src_v)
            pltpu.sync_copy(dst_hbm.at[s].at[pl.ds(p * CPP, CPP)], dst_v)
            pltpu.async_copy(x_view.at[src_v.at[0]], rows0_v, sem0)

            @pl.loop(0, CPP // 2)
            def _(k):
                j0 = 2 * k
                g1 = pltpu.async_copy(x_view.at[src_v.at[j0 + 1]], rows1_v,
                                      sem1)
                pltpu.make_async_copy(x_view.at[src_v.at[j0]], rows0_v,
                                      sem0).wait()
                pltpu.sync_copy(rows0_v, acc.at[dst_v.at[j0]], add=True)

                @pl.when(j0 + 2 < CPP)
                def _():
                    pltpu.async_copy(x_view.at[src_v.at[j0 + 2]], rows0_v,
                                     sem0)

                g1.wait()
                pltpu.sync_copy(rows1_v, acc.at[dst_v.at[j0 + 1]], add=True)

        plsc.subcore_barrier()
        write_out(sums_hbm)
        plsc.subcore_barrier()

        zero_acc()
        pltpu.sync_copy(ones_hbm, rows0_v)
        pltpu.sync_copy(dst_hbm.at[s].at[pl.ds(c * CPP, CPP)], dst_v)
        plsc.subcore_barrier()

        @pl.loop(0, CPP)
        def _(j):
            pltpu.sync_copy(rows0_v, acc.at[dst_v.at[j]], add=True)

        plsc.subcore_barrier()
        write_out(cnt_hbm)

    return agg_kernel(xt, dst_r, src_r, zrow, ones)


def _tc_combine(features, sums2, counts, weight):
    def body(x_ref, s_ref, c_ref, w_ref, o_ref):
        w = w_ref[...]
        nodes = jnp.dot(x_ref[...], w, preferred_element_type=jnp.float32)
        agg = jnp.concatenate([s_ref[0], s_ref[1]], axis=-1)
        cnt = (c_ref[0] + c_ref[1])[:, :1]
        agg = agg / jnp.maximum(cnt, 1.0)
        msgs = jnp.dot(agg, w, preferred_element_type=jnp.float32)
        o_ref[...] = jnp.concatenate([nodes, msgs], axis=-1)

    return pl.pallas_call(
        body,
        grid=(N_NODES // BLK,),
        in_specs=[
            pl.BlockSpec((BLK, FEAT), lambda i: (i, 0)),
            pl.BlockSpec((2, BLK, HALF), lambda i: (0, i, 0)),
            pl.BlockSpec((2, BLK, CNT_W), lambda i: (0, i, 0)),
            pl.BlockSpec((FEAT, FEAT), lambda i: (0, 0)),
        ],
        out_specs=pl.BlockSpec((BLK, 2 * FEAT), lambda i: (i, 0)),
        out_shape=jax.ShapeDtypeStruct((N_NODES, 2 * FEAT), jnp.float32),
    )(features, sums2, counts, weight)


def kernel(features, edge_index, weight):
    dst_r = edge_index[0].reshape(NSUB, NCHUNK, CHUNK)
    src_r = edge_index[1].reshape(NSUB, NCHUNK, CHUNK)
    zrow = jnp.zeros((ROWS_PER_TILE, HALF), jnp.float32)
    ones = jnp.ones((CHUNK, HALF), jnp.float32)
    sums2, counts = _sc_aggregate(features, dst_r, src_r, zrow, ones)
    return _tc_combine(features, sums2, counts, weight)
